# baseline (speedup 1.0000x reference)
import functools

import jax
import jax.numpy as jnp
from jax.experimental import pallas as pl
from jax.experimental.pallas import tpu as pltpu

_TN = 512


def _chamfer_tile_kernel(px_ref, ty_ref, xmin_ref, ymin_ref):
    i = pl.program_id(1)
    px = px_ref[0]
    ty = ty_ref[0]
    xx = jnp.sum(px * px, axis=1, keepdims=True)
    yy = jnp.sum(ty * ty, axis=0, keepdims=True)
    xy = jnp.dot(
        px.astype(jnp.bfloat16),
        ty.astype(jnp.bfloat16),
        preferred_element_type=jnp.float32,
    )
    d = xx + yy - 2.0 * xy
    xmin_ref[0, 0, 0, :] = jnp.min(d, axis=1)
    ymin_tile = jnp.min(d, axis=0, keepdims=True)[None]

    @pl.when(i == 0)
    def _init():
        ymin_ref[...] = ymin_tile

    @pl.when(i > 0)
    def _acc():
        ymin_ref[...] = jnp.minimum(ymin_ref[...], ymin_tile)


@functools.partial(jax.jit, static_argnames=())
def _chamfer(predict, target):
    B, N, _ = predict.shape
    _, M, _ = target.shape
    ty = target.transpose(0, 2, 1)
    nb = N // _TN
    x_near, y_near = pl.pallas_call(
        _chamfer_tile_kernel,
        grid=(B, nb),
        in_specs=[
            pl.BlockSpec((1, _TN, 3), lambda b, i: (b, i, 0)),
            pl.BlockSpec((1, 3, M), lambda b, i: (b, 0, 0)),
        ],
        out_specs=[
            pl.BlockSpec((1, 1, 1, _TN), lambda b, i: (b, i, 0, 0)),
            pl.BlockSpec((1, 1, M), lambda b, i: (b, 0, 0)),
        ],
        out_shape=[
            jax.ShapeDtypeStruct((B, nb, 1, _TN), jnp.float32),
            jax.ShapeDtypeStruct((B, 1, M), jnp.float32),
        ],
        compiler_params=pltpu.CompilerParams(
            dimension_semantics=("parallel", "arbitrary"),
        ),
    )(predict, ty)
    return x_near.mean() + y_near.mean()


def kernel(predict, target):
    return _chamfer(predict, target)
